# baseline (speedup 1.0000x reference)
import functools

import jax
import jax.numpy as jnp
from jax import lax
from jax.experimental import pallas as pl
from jax.experimental.pallas import tpu as pltpu
from jax.experimental.pallas import tpu_sc as plsc

L = 16
NC = 2
NS = 16
NW = NC * NS
BB = 64
PB = 2


def _allsum_vec(v):
    idx = lax.iota(jnp.int32, L)
    dnums = lax.GatherDimensionNumbers(
        offset_dims=(), collapsed_slice_dims=(0,), start_index_map=(0,))
    for d in (1, 2, 4, 8):
        v = v + lax.gather(
            v, (idx ^ d)[:, None], dimension_numbers=dnums, slice_sizes=(1,),
            mode=lax.GatherScatterMode.PROMISE_IN_BOUNDS)
    return v


def _rsqrt_vec(v):
    i = lax.bitcast_convert_type(v, jnp.int32)
    magic = jnp.full((L,), 0x5F3759DF, jnp.int32)
    y = lax.bitcast_convert_type(magic - (i >> 1), jnp.float32)
    for _ in range(3):
        y = y * (1.5 - 0.5 * v * y * y)
    return y


def _make_sc_kernel(B, S, V, D):
    upw = S * PB
    tpw = upw * BB
    nvec = D // L
    mesh = plsc.VectorSubcoreMesh(
        core_axis_name="c", subcore_axis_name="s", num_cores=NC, num_subcores=NS
    )

    @functools.partial(
        pl.kernel,
        out_type=jax.ShapeDtypeStruct((S, B, D), jnp.float32),
        mesh=mesh,
        compiler_params=pltpu.CompilerParams(use_tc_tiling_on_sc=False),
        scratch_types=[
            pltpu.VMEM((tpw,), jnp.int32),
            pltpu.VMEM((D,), jnp.float32),
            pltpu.VMEM((D,), jnp.float32),
            pltpu.VMEM((D,), jnp.float32),
            [pltpu.VMEM((BB, D), jnp.float32)] * 2,
            pltpu.SMEM((BB,), jnp.float32),
            pltpu.SMEM((BB,), jnp.float32),
            [pltpu.SemaphoreType.DMA] * 2,
            [pltpu.SemaphoreType.DMA] * 2,
        ],
    )
    def k(xu_ref, tok_ref, pos_ref, gamma_ref, beta_ref, out_ref,
          idx_v, pos_row, gamma_v, beta_v, bufs, mean_s, rstd_s, gsems, osems):
        wid = lax.axis_index("s") * NC + lax.axis_index("c")

        pltpu.sync_copy(xu_ref.at[pl.ds(wid * tpw, tpw)], idx_v)
        pltpu.sync_copy(gamma_ref, gamma_v)
        pltpu.sync_copy(beta_ref, beta_v)

        def issue_gather(u, b):
            pltpu.async_copy(
                tok_ref.at[idx_v.at[pl.ds(u * BB, BB)]], bufs[b], gsems[b])

        def load_pos(s):
            pltpu.sync_copy(
                pos_ref.at[0, pl.ds(pl.multiple_of(s * D, D), D)], pos_row)

        def pass1(buf):
            @plsc.parallel_loop(0, BB, unroll=2)
            def row_body(r):
                sacc = [jnp.zeros((L,), jnp.float32) for _ in range(6)]
                qacc = [jnp.zeros((L,), jnp.float32) for _ in range(6)]
                for j in range(nvec):
                    v = buf[r, pl.ds(j * L, L)] + pos_row[pl.ds(j * L, L)]
                    buf[r, pl.ds(j * L, L)] = v
                    sacc[j % 6] = sacc[j % 6] + v
                    qacc[j % 6] = qacc[j % 6] + v * v
                s = ((sacc[0] + sacc[1]) + (sacc[2] + sacc[3])) + (sacc[4] + sacc[5])
                q = ((qacc[0] + qacc[1]) + (qacc[2] + qacc[3])) + (qacc[4] + qacc[5])
                mean_v = _allsum_vec(s) * (1.0 / D)
                msq_v = _allsum_vec(q) * (1.0 / D)
                var_v = msq_v - mean_v * mean_v
                rstd_v = _rsqrt_vec(var_v + 1e-5)
                mean_s[r] = mean_v[0]
                rstd_s[r] = rstd_v[0]

        def pass2(buf):
            def col_body(j, _):
                g_vec = gamma_v[pl.ds(j * L, L)]
                b_vec = beta_v[pl.ds(j * L, L)]

                @plsc.parallel_loop(0, BB, unroll=2)
                def row_body(r):
                    v = buf[r, pl.ds(j * L, L)]
                    t = (v - mean_s[r]) * rstd_s[r]
                    buf[r, pl.ds(j * L, L)] = t * g_vec + b_vec

                return 0

            lax.fori_loop(0, nvec, col_body, 0)

        issue_gather(0, 0)

        def s_step(s):
            load_pos(s)
            for b in range(PB):
                u = s * PB + b
                bb = wid * PB + b
                dst = out_ref.at[s, pl.ds(pl.multiple_of(bb * BB, BB), BB)]
                pltpu.make_async_copy(
                    tok_ref.at[idx_v.at[pl.ds(u * BB, BB)]],
                    bufs[b], gsems[b]).wait()
                pass1(bufs[b])
                @pl.when(u >= 1)
                def _():
                    pltpu.make_async_copy(
                        bufs[1 - b], out_ref.at[0, pl.ds(0, BB)],
                        osems[1 - b]).wait()

                @pl.when(u + 1 < upw)
                def _():
                    issue_gather(u + 1, 1 - b)

                pass2(bufs[b])
                pltpu.async_copy(bufs[b], dst, osems[b])

        def outer(s, _):
            s_step(s)
            return 0

        lax.fori_loop(0, S, outer, 0)
        b_last = (upw - 1) % 2
        pltpu.make_async_copy(
            bufs[b_last], out_ref.at[0, pl.ds(0, BB)], osems[b_last]).wait()

    return k


def kernel(x, tok_table, pos_table, gamma, beta):
    B, S = x.shape
    V, D = tok_table.shape
    nbb = B // BB
    xu = (x.T.reshape(S, nbb // PB, PB, BB)
          .transpose(1, 0, 2, 3).reshape(-1))
    pos_flat = pos_table[:S].reshape(1, S * D)
    k = _make_sc_kernel(B, S, V, D)
    out = k(xu, tok_table, pos_flat, gamma, beta)
    return out.transpose(1, 0, 2)


# speedup vs baseline: 2.5662x; 2.5662x over previous
import functools

import jax
import jax.numpy as jnp
from jax import lax
from jax.experimental import pallas as pl
from jax.experimental.pallas import tpu as pltpu
from jax.experimental.pallas import tpu_sc as plsc

NC = 2
NS = 16
NW = NC * NS
K = 4
BR = 8


@functools.cache
def _make_gather_kernel(S, V, D, BCH):
    BB = BCH // NW
    tpw = S * BB
    mesh = plsc.VectorSubcoreMesh(
        core_axis_name="c", subcore_axis_name="s", num_cores=NC, num_subcores=NS
    )

    @functools.partial(
        pl.kernel,
        out_type=jax.ShapeDtypeStruct((S, BCH, D), jnp.float32),
        mesh=mesh,
        scratch_types=[
            pltpu.VMEM((tpw,), jnp.int32),
            [pltpu.VMEM((BB, D), jnp.float32)] * 2,
            [pltpu.SemaphoreType.DMA] * 2,
            [pltpu.SemaphoreType.DMA] * 2,
        ],
    )
    def gk(xu_ref, tok_ref, emb_ref, idx_v, bufs, gsems, osems):
        wid = lax.axis_index("s") * NC + lax.axis_index("c")
        col = pl.multiple_of(wid * BB, BB)

        pltpu.sync_copy(xu_ref.at[pl.ds(wid * tpw, tpw)], idx_v)

        def issue_gather(s, b):
            pltpu.async_copy(
                tok_ref.at[idx_v.at[pl.ds(s * BB, BB)]], bufs[b], gsems[b])

        issue_gather(0, 0)

        def s_step(s, b):
            pltpu.make_async_copy(
                tok_ref.at[idx_v.at[pl.ds(s * BB, BB)]],
                bufs[b], gsems[b]).wait()
            pltpu.async_copy(bufs[b], emb_ref.at[s, pl.ds(col, BB)], osems[b])

            @pl.when(s >= 1)
            def _():
                pltpu.make_async_copy(
                    bufs[1 - b], emb_ref.at[0, pl.ds(0, BB)],
                    osems[1 - b]).wait()

            @pl.when(s + 1 < S)
            def _():
                issue_gather(s + 1, 1 - b)

        def outer(o, _):
            for b in range(2):
                s_step(o * 2 + b, b)
            return 0

        lax.fori_loop(0, S // 2, outer, 0)
        b_last = (S - 1) % 2
        pltpu.make_async_copy(
            bufs[b_last], emb_ref.at[0, pl.ds(0, BB)], osems[b_last]).wait()

    return gk


def _ln_body(emb_ref, pos_ref, g_ref, b_ref, *rest):
    out_ref = rest[-1]
    S = emb_ref.shape[0]
    e = emb_ref[...] + pos_ref[0:S, :][:, None, :]
    mean = jnp.mean(e, axis=2, keepdims=True)
    c = e - mean
    var = jnp.mean(c * c, axis=2, keepdims=True)
    out_ref[...] = (c * lax.rsqrt(var + 1e-5) * g_ref[0][None, None, :]
                    + b_ref[0][None, None, :])


def _make_ln_call(k_idx, S, B, D, BCH, SP, aliased):
    nblk = BCH // BR
    base = k_idx * nblk
    out_spec = pl.BlockSpec((S, BR, D), lambda g: (0, base + g, 0))
    in_specs = [
        pl.BlockSpec((S, BR, D), lambda g: (0, g, 0)),
        pl.BlockSpec((SP, D), lambda g: (0, 0)),
        pl.BlockSpec((1, D), lambda g: (0, 0)),
        pl.BlockSpec((1, D), lambda g: (0, 0)),
    ]
    kwargs = {}
    if aliased:
        in_specs.append(pl.BlockSpec(memory_space=pl.ANY))
        kwargs["input_output_aliases"] = {4: 0}
    return pl.pallas_call(
        _ln_body,
        grid=(nblk,),
        in_specs=in_specs,
        out_specs=out_spec,
        out_shape=jax.ShapeDtypeStruct((S, B, D), jnp.float32),
        **kwargs,
    )


def kernel(x, tok_table, pos_table, gamma, beta):
    B, S = x.shape
    V, D = tok_table.shape
    SP = pos_table.shape[0]
    BCH = B // K
    BB = BCH // NW
    xu = x.reshape(K, NW, BB, S).transpose(0, 1, 3, 2).reshape(K, -1)
    g2 = gamma.reshape(1, D)
    b2 = beta.reshape(1, D)
    gk = _make_gather_kernel(S, V, D, BCH)
    out = None
    for k in range(K):
        emb = gk(xu[k], tok_table)
        ln = _make_ln_call(k, S, B, D, BCH, SP, aliased=k > 0)
        args = (emb, pos_table, g2, b2) + ((out,) if k > 0 else ())
        out = ln(*args)
    return out.transpose(1, 0, 2)


# speedup vs baseline: 2.6257x; 1.0232x over previous
import functools

import jax
import jax.numpy as jnp
from jax import lax
from jax.experimental import pallas as pl
from jax.experimental.pallas import tpu as pltpu
from jax.experimental.pallas import tpu_sc as plsc

NC = 2
NS = 16
NW = NC * NS
K = 8
BR = 8


@functools.cache
def _make_gather_kernel(S, V, D, BCH):
    BB = BCH // NW
    tpw = S * BB
    mesh = plsc.VectorSubcoreMesh(
        core_axis_name="c", subcore_axis_name="s", num_cores=NC, num_subcores=NS
    )

    @functools.partial(
        pl.kernel,
        out_type=jax.ShapeDtypeStruct((S, BCH, D), jnp.float32),
        mesh=mesh,
        scratch_types=[
            pltpu.VMEM((tpw,), jnp.int32),
            [pltpu.VMEM((BB, D), jnp.float32)] * 2,
            [pltpu.SemaphoreType.DMA] * 2,
            [pltpu.SemaphoreType.DMA] * 2,
        ],
    )
    def gk(xu_ref, tok_ref, emb_ref, idx_v, bufs, gsems, osems):
        wid = lax.axis_index("s") * NC + lax.axis_index("c")
        col = pl.multiple_of(wid * BB, BB)

        pltpu.sync_copy(xu_ref.at[pl.ds(wid * tpw, tpw)], idx_v)

        def issue_gather(s, b):
            pltpu.async_copy(
                tok_ref.at[idx_v.at[pl.ds(s * BB, BB)]], bufs[b], gsems[b])

        issue_gather(0, 0)

        def s_step(s, b):
            pltpu.make_async_copy(
                tok_ref.at[idx_v.at[pl.ds(s * BB, BB)]],
                bufs[b], gsems[b]).wait()
            pltpu.async_copy(bufs[b], emb_ref.at[s, pl.ds(col, BB)], osems[b])

            @pl.when(s >= 1)
            def _():
                pltpu.make_async_copy(
                    bufs[1 - b], emb_ref.at[0, pl.ds(0, BB)],
                    osems[1 - b]).wait()

            @pl.when(s + 1 < S)
            def _():
                issue_gather(s + 1, 1 - b)

        def outer(o, _):
            for b in range(2):
                s_step(o * 2 + b, b)
            return 0

        lax.fori_loop(0, S // 2, outer, 0)
        b_last = (S - 1) % 2
        pltpu.make_async_copy(
            bufs[b_last], emb_ref.at[0, pl.ds(0, BB)], osems[b_last]).wait()

    return gk


def _ln_body(emb_ref, pos_ref, g_ref, b_ref, *rest):
    out_ref = rest[-1]
    S = emb_ref.shape[0]
    e = emb_ref[...] + pos_ref[0:S, :][:, None, :]
    mean = jnp.mean(e, axis=2, keepdims=True)
    c = e - mean
    var = jnp.mean(c * c, axis=2, keepdims=True)
    out_ref[...] = (c * lax.rsqrt(var + 1e-5) * g_ref[0][None, None, :]
                    + b_ref[0][None, None, :])


def _make_ln_call(k_idx, S, B, D, BCH, SP, aliased):
    nblk = BCH // BR
    base = k_idx * nblk
    out_spec = pl.BlockSpec((S, BR, D), lambda g: (0, base + g, 0))
    in_specs = [
        pl.BlockSpec((S, BR, D), lambda g: (0, g, 0)),
        pl.BlockSpec((SP, D), lambda g: (0, 0)),
        pl.BlockSpec((1, D), lambda g: (0, 0)),
        pl.BlockSpec((1, D), lambda g: (0, 0)),
    ]
    kwargs = {}
    if aliased:
        in_specs.append(pl.BlockSpec(memory_space=pl.ANY))
        kwargs["input_output_aliases"] = {4: 0}
    return pl.pallas_call(
        _ln_body,
        grid=(nblk,),
        in_specs=in_specs,
        out_specs=out_spec,
        out_shape=jax.ShapeDtypeStruct((S, B, D), jnp.float32),
        **kwargs,
    )


def kernel(x, tok_table, pos_table, gamma, beta):
    B, S = x.shape
    V, D = tok_table.shape
    SP = pos_table.shape[0]
    BCH = B // K
    BB = BCH // NW
    xu = x.reshape(K, NW, BB, S).transpose(0, 1, 3, 2).reshape(K, -1)
    g2 = gamma.reshape(1, D)
    b2 = beta.reshape(1, D)
    gk = _make_gather_kernel(S, V, D, BCH)
    out = None
    for k in range(K):
        emb = gk(xu[k], tok_table)
        ln = _make_ln_call(k, S, B, D, BCH, SP, aliased=k > 0)
        args = (emb, pos_table, g2, b2) + ((out,) if k > 0 else ())
        out = ln(*args)
    return out.transpose(1, 0, 2)


# speedup vs baseline: 2.6311x; 1.0020x over previous
import functools

import jax
import jax.numpy as jnp
from jax import lax
from jax.experimental import pallas as pl
from jax.experimental.pallas import tpu as pltpu
from jax.experimental.pallas import tpu_sc as plsc

NC = 2
NS = 16
NW = NC * NS
K = 8
BR = 8


@functools.cache
def _make_gather_kernel(S, V, D, BCH):
    BB = BCH // NW
    tpw = S * BB
    mesh = plsc.VectorSubcoreMesh(
        core_axis_name="c", subcore_axis_name="s", num_cores=NC, num_subcores=NS
    )

    @functools.partial(
        pl.kernel,
        out_type=jax.ShapeDtypeStruct((S, BCH, D), jnp.float32),
        mesh=mesh,
        scratch_types=[
            pltpu.VMEM((tpw,), jnp.int32),
            [pltpu.VMEM((BB, D), jnp.float32)] * 2,
            [pltpu.SemaphoreType.DMA] * 2,
            [pltpu.SemaphoreType.DMA] * 2,
        ],
    )
    def gk(xu_ref, tok_ref, emb_ref, idx_v, bufs, gsems, osems):
        wid = lax.axis_index("s") * NC + lax.axis_index("c")
        col = pl.multiple_of(wid * BB, BB)

        pltpu.sync_copy(xu_ref.at[pl.ds(wid * tpw, tpw)], idx_v)

        def issue_gather(s, b):
            pltpu.async_copy(
                tok_ref.at[idx_v.at[pl.ds(s * BB, BB)]], bufs[b], gsems[b])

        issue_gather(0, 0)

        def s_step(s, b):
            pltpu.make_async_copy(
                tok_ref.at[idx_v.at[pl.ds(s * BB, BB)]],
                bufs[b], gsems[b]).wait()
            pltpu.async_copy(bufs[b], emb_ref.at[s, pl.ds(col, BB)], osems[b])

            @pl.when(s >= 1)
            def _():
                pltpu.make_async_copy(
                    bufs[1 - b], emb_ref.at[0, pl.ds(0, BB)],
                    osems[1 - b]).wait()

            @pl.when(s + 1 < S)
            def _():
                issue_gather(s + 1, 1 - b)

        def outer(o, _):
            for b in range(2):
                s_step(o * 2 + b, b)
            return 0

        lax.fori_loop(0, S // 2, outer, 0)
        b_last = (S - 1) % 2
        pltpu.make_async_copy(
            bufs[b_last], emb_ref.at[0, pl.ds(0, BB)], osems[b_last]).wait()

    return gk


def _ln_body(emb_ref, pos_ref, g_ref, b_ref, *rest):
    out_ref = rest[-1]
    S = emb_ref.shape[0]
    e = emb_ref[...] + pos_ref[0:S, :][:, None, :]
    mean = jnp.mean(e, axis=2, keepdims=True)
    c = e - mean
    var = jnp.mean(c * c, axis=2, keepdims=True)
    out_ref[...] = (c * lax.rsqrt(var + 1e-5) * g_ref[0][None, None, :]
                    + b_ref[0][None, None, :])


def _make_ln_call(k_idx, S, B, D, BCH, SP, aliased):
    nblk = BCH // BR
    base = k_idx * nblk
    out_spec = pl.BlockSpec((S, BR, D), lambda g: (0, base + g, 0))
    in_specs = [
        pl.BlockSpec((S, BR, D), lambda g: (0, g, 0)),
        pl.BlockSpec((SP, D), lambda g: (0, 0)),
        pl.BlockSpec((1, D), lambda g: (0, 0)),
        pl.BlockSpec((1, D), lambda g: (0, 0)),
    ]
    kwargs = {}
    if aliased:
        in_specs.append(pl.BlockSpec(memory_space=pl.ANY))
        kwargs["input_output_aliases"] = {4: 0}
    return pl.pallas_call(
        _ln_body,
        grid=(nblk,),
        in_specs=in_specs,
        out_specs=out_spec,
        out_shape=jax.ShapeDtypeStruct((S, B, D), jnp.float32),
        **kwargs,
    )


def kernel(x, tok_table, pos_table, gamma, beta):
    B, S = x.shape
    V, D = tok_table.shape
    SP = pos_table.shape[0]
    BCH = B // K
    BB = BCH // NW
    g2 = gamma.reshape(1, D)
    b2 = beta.reshape(1, D)
    gk = _make_gather_kernel(S, V, D, BCH)
    out = None
    for k in range(K):
        xu = (x[k * BCH:(k + 1) * BCH].reshape(NW, BB, S)
              .transpose(0, 2, 1).reshape(-1))
        emb = gk(xu, tok_table)
        ln = _make_ln_call(k, S, B, D, BCH, SP, aliased=k > 0)
        args = (emb, pos_table, g2, b2) + ((out,) if k > 0 else ())
        out = ln(*args)
    return out.transpose(1, 0, 2)
